# baseline (speedup 1.0000x reference)
import functools

import jax
import jax.numpy as jnp
from jax import lax
from jax.experimental import pallas as pl
from jax.experimental.pallas import tpu as pltpu
from jax.experimental.pallas import tpu_sc as plsc

N_NODES = 10000
D = 128

NC = 2
NS = 16
NWK = NC * NS

CHUNK = 128
PACK = 14
N_PAD = 10112
BS2 = 632


def _make_sc_scatter(e_pad: int, with_deg: bool):
    npw = e_pad // NWK
    nch = npw // CHUNK
    assert npw % CHUNK == 0 and nch % 2 == 0
    rpt = N_PAD // NS

    mesh = plsc.VectorSubcoreMesh(
        core_axis_name="c", subcore_axis_name="s", num_cores=NC, num_subcores=NS)

    out_type = [jax.ShapeDtypeStruct((NC, N_PAD, D), jnp.float32)]
    scratch = [
        pltpu.VMEM((npw,), jnp.int32),
        pltpu.VMEM((2, CHUNK), jnp.int32),
        pltpu.VMEM((2, CHUNK), jnp.int32),
        pltpu.VMEM((2, CHUNK, D), jnp.float32),
        pltpu.VMEM_SHARED((N_PAD, D), jnp.float32),
        pltpu.SemaphoreType.DMA,
        pltpu.SemaphoreType.DMA,
        pltpu.SemaphoreType.DMA,
        pltpu.SemaphoreType.DMA,
    ]
    if with_deg:
        out_type.append(jax.ShapeDtypeStruct((NC, N_PAD), jnp.float32))
        scratch += [
            pltpu.VMEM_SHARED((N_PAD,), jnp.float32),
            pltpu.VMEM((CHUNK,), jnp.float32),
        ]

    @functools.partial(
        pl.kernel,
        out_type=out_type,
        mesh=mesh,
        scratch_types=scratch,
        compiler_params=pltpu.CompilerParams(use_tc_tiling_on_sc=False),
    )
    def sc_kernel(tab, pkh, zer, zer1, *refs):
        if with_deg:
            (out, outd, pidx, colb, rowb, rowsv, acc,
             g0, g1, s0, s1, accd, ones) = refs
        else:
            out, pidx, colb, rowb, rowsv, acc, g0, g1, s0, s1 = refs
        core = lax.axis_index("c")
        sub = lax.axis_index("s")
        wid = core * NS + sub
        r0 = sub * rpt

        pltpu.sync_copy(zer.at[pl.ds(r0, rpt)], acc.at[pl.ds(r0, rpt)])
        pltpu.sync_copy(pkh.at[pl.ds(wid * npw, npw)], pidx)
        if with_deg:
            @pl.when(sub == 0)
            def _zero_deg():
                pltpu.sync_copy(zer1, accd)
            for k in range(CHUNK // 16):
                ones[pl.ds(16 * k, 16)] = jnp.ones((16,), jnp.float32)
        plsc.subcore_barrier()

        gsem = (g0, g1)
        ssem = (s0, s1)

        def unpack(j, b):
            base = j * CHUNK
            for k in range(CHUNK // 16):
                pk = pidx[pl.ds(base + 16 * k, 16)]
                rowb[b, pl.ds(16 * k, 16)] = lax.shift_right_logical(pk, PACK)
                colb[b, pl.ds(16 * k, 16)] = lax.bitwise_and(
                    pk, (1 << PACK) - 1)

        def start_gather(b):
            pltpu.async_copy(tab.at[colb.at[b]], rowsv.at[b], gsem[b])

        def wait_gather(b):
            pltpu.make_async_copy(tab.at[colb.at[b]], rowsv.at[b],
                                  gsem[b]).wait()

        def start_scatter(b):
            pltpu.async_copy(rowsv.at[b], acc.at[rowb.at[b]], ssem[b],
                             add=True)
            if with_deg:
                pltpu.async_copy(ones, accd.at[rowb.at[b]], ssem[b], add=True)

        def wait_scatter(b):
            pltpu.make_async_copy(rowsv.at[b], acc.at[rowb.at[b]],
                                  ssem[b]).wait()
            if with_deg:
                pltpu.make_async_copy(ones, accd.at[rowb.at[b]],
                                      ssem[b]).wait()

        unpack(0, 0)
        start_gather(0)

        def body(i, _):
            g = 2 * i
            wait_gather(0)
            start_scatter(0)

            @pl.when(g > 0)
            def _drain_prev_odd():
                wait_scatter(1)

            unpack(g + 1, 1)
            start_gather(1)
            wait_gather(1)
            start_scatter(1)
            wait_scatter(0)

            @pl.when(g + 2 < nch)
            def _prefetch():
                unpack(g + 2, 0)
                start_gather(0)

            return ()

        lax.fori_loop(0, nch // 2, body, ())
        wait_scatter(1)

        plsc.subcore_barrier()
        pltpu.sync_copy(acc.at[pl.ds(r0, rpt)], out.at[core].at[pl.ds(r0, rpt)])
        if with_deg:
            @pl.when(sub == 0)
            def _pub_deg():
                pltpu.sync_copy(accd, outd.at[core])

    return sc_kernel


def _rdeg_tc(cnt2d):
    nr = N_PAD // D

    def body(c_ref, rd_ref):
        rd_ref[...] = 1.0 / jnp.maximum(c_ref[0] + c_ref[1], 1.0)

    return pl.pallas_call(
        body,
        out_shape=jax.ShapeDtypeStruct((nr, D), jnp.float32),
    )(cnt2d)


def _mlp_tc(p, rdcol, wt, b):
    grid = N_PAD // BS2

    def body(p_ref, rd_ref, w_ref, b_ref, o_ref):
        s = (p_ref[0] + p_ref[1]) * rd_ref[...]
        o = jnp.dot(s, w_ref[...], preferred_element_type=jnp.float32)
        o_ref[...] = jnp.maximum(o + b_ref[...], 0.0)

    return pl.pallas_call(
        body,
        grid=(grid,),
        in_specs=[
            pl.BlockSpec((NC, BS2, D), lambda i: (0, i, 0)),
            pl.BlockSpec((BS2, 1), lambda i: (i, 0)),
            pl.BlockSpec((D, D), lambda i: (0, 0)),
            pl.BlockSpec((1, D), lambda i: (0, 0)),
        ],
        out_specs=pl.BlockSpec((BS2, D), lambda i: (i, 0)),
        out_shape=jax.ShapeDtypeStruct((N_PAD, D), jnp.float32),
    )(p, rdcol, wt, b)


@jax.jit
def kernel(x, edge_index, W1, b1, W2, b2):
    n = x.shape[0]
    e = edge_index.shape[1]

    eif = edge_index.reshape(2 * e).astype(jnp.int32)
    packed = lax.shift_left(eif[:e], PACK) | eif[e:]

    e_pad = -(-e // (NWK * 2 * CHUNK)) * (NWK * 2 * CHUNK)
    npad = e_pad - e
    pad_rows = n + (jnp.arange(npad, dtype=jnp.int32) % (N_PAD - n))
    pad_cols = jnp.arange(npad, dtype=jnp.int32) % n
    packed_p = jnp.concatenate(
        [packed, lax.shift_left(pad_rows, PACK) | pad_cols])

    z_d = jnp.zeros((N_PAD, D), dtype=jnp.float32)
    z_1 = jnp.zeros((N_PAD,), dtype=jnp.float32)

    sc1 = _make_sc_scatter(e_pad, with_deg=True)
    sc2 = _make_sc_scatter(e_pad, with_deg=False)

    p1, cnt = sc1(x, packed_p, z_d, z_1)
    rd2 = _rdeg_tc(cnt.reshape(NC, N_PAD // D, D))
    rdcol = rd2.reshape(N_PAD, 1)
    h = _mlp_tc(p1, rdcol, W1.T, b1.reshape(1, D))

    (p2,) = sc2(h, packed_p, z_d, z_1)
    out = _mlp_tc(p2, rdcol, W2.T, b2.reshape(1, D))

    return out[:n]


# speedup vs baseline: 1.1771x; 1.1771x over previous
import functools

import jax
import jax.numpy as jnp
from jax import lax
from jax.experimental import pallas as pl
from jax.experimental.pallas import tpu as pltpu
from jax.experimental.pallas import tpu_sc as plsc

N_NODES = 10000
D = 128

NC = 2
NS = 16
NWK = NC * NS

CHUNK = 64
NBUF = 4
PACK = 14
N_PAD = 10112
BS2 = 632


def _make_sc_scatter(e_pad: int, with_deg: bool):
    npw = e_pad // NWK
    nch = npw // CHUNK
    assert npw % CHUNK == 0 and nch % NBUF == 0
    rpt = N_PAD // NS

    mesh = plsc.VectorSubcoreMesh(
        core_axis_name="c", subcore_axis_name="s", num_cores=NC, num_subcores=NS)

    out_type = [jax.ShapeDtypeStruct((NC, N_PAD, D), jnp.float32)]
    scratch = [
        pltpu.VMEM((npw,), jnp.int32),
        pltpu.VMEM((NBUF, CHUNK), jnp.int32),
        pltpu.VMEM((NBUF, CHUNK), jnp.int32),
        pltpu.VMEM((NBUF, CHUNK, D), jnp.float32),
        pltpu.VMEM_SHARED((N_PAD, D), jnp.float32),
        [pltpu.SemaphoreType.DMA] * NBUF,
        [pltpu.SemaphoreType.DMA] * NBUF,
    ]
    if with_deg:
        out_type.append(jax.ShapeDtypeStruct((NC, N_PAD), jnp.float32))
        scratch += [
            pltpu.VMEM_SHARED((N_PAD,), jnp.float32),
            pltpu.VMEM((CHUNK,), jnp.float32),
        ]

    @functools.partial(
        pl.kernel,
        out_type=out_type,
        mesh=mesh,
        scratch_types=scratch,
        compiler_params=pltpu.CompilerParams(use_tc_tiling_on_sc=False),
    )
    def sc_kernel(tab, pkh, zer, zer1, *refs):
        if with_deg:
            (out, outd, pidx, colb, rowb, rowsv, acc,
             gsem, ssem, accd, ones) = refs
        else:
            out, pidx, colb, rowb, rowsv, acc, gsem, ssem = refs
        core = lax.axis_index("c")
        sub = lax.axis_index("s")
        wid = core * NS + sub
        r0 = sub * rpt

        pltpu.sync_copy(zer.at[pl.ds(r0, rpt)], acc.at[pl.ds(r0, rpt)])
        pltpu.sync_copy(pkh.at[pl.ds(wid * npw, npw)], pidx)
        if with_deg:
            @pl.when(sub == 0)
            def _zero_deg():
                pltpu.sync_copy(zer1, accd)
            for k in range(CHUNK // 16):
                ones[pl.ds(16 * k, 16)] = jnp.ones((16,), jnp.float32)
        plsc.subcore_barrier()

        def unpack(j, b):
            base = j * CHUNK
            for k in range(CHUNK // 16):
                pk = pidx[pl.ds(base + 16 * k, 16)]
                rowb[b, pl.ds(16 * k, 16)] = lax.shift_right_logical(pk, PACK)
                colb[b, pl.ds(16 * k, 16)] = lax.bitwise_and(
                    pk, (1 << PACK) - 1)

        def start_gather(b):
            pltpu.async_copy(tab.at[colb.at[b]], rowsv.at[b], gsem[b])

        def wait_gather(b):
            pltpu.make_async_copy(tab.at[colb.at[b]], rowsv.at[b],
                                  gsem[b]).wait()

        def start_scatter(b):
            pltpu.async_copy(rowsv.at[b], acc.at[rowb.at[b]], ssem[b],
                             add=True)
            if with_deg:
                pltpu.async_copy(ones, accd.at[rowb.at[b]], ssem[b], add=True)

        def wait_scatter(b):
            pltpu.make_async_copy(rowsv.at[b], acc.at[rowb.at[b]],
                                  ssem[b]).wait()
            if with_deg:
                pltpu.make_async_copy(ones, accd.at[rowb.at[b]],
                                      ssem[b]).wait()

        for j in range(NBUF - 1):
            unpack(j, j)
            start_gather(j)

        def body(i, _):
            g = NBUF * i
            for t in range(NBUF):
                j = g + t
                wait_gather(t)
                start_scatter(t)
                bp = (t - 1) % NBUF

                @pl.when(j + NBUF - 1 < nch)
                def _prefetch(j=j, t=t, bp=bp):
                    @pl.when(j > 0)
                    def _drain():
                        wait_scatter(bp)

                    unpack(j + NBUF - 1, bp)
                    start_gather(bp)
            return ()

        lax.fori_loop(0, nch // NBUF, body, ())
        for t in range(NBUF):
            wait_scatter(t)

        plsc.subcore_barrier()
        pltpu.sync_copy(acc.at[pl.ds(r0, rpt)], out.at[core].at[pl.ds(r0, rpt)])
        if with_deg:
            @pl.when(sub == 0)
            def _pub_deg():
                pltpu.sync_copy(accd, outd.at[core])

    return sc_kernel


def _rdeg_tc(cnt2d):
    nr = N_PAD // D

    def body(c_ref, rd_ref):
        rd_ref[...] = 1.0 / jnp.maximum(c_ref[0] + c_ref[1], 1.0)

    return pl.pallas_call(
        body,
        out_shape=jax.ShapeDtypeStruct((nr, D), jnp.float32),
    )(cnt2d)


def _mlp_tc(p, rdcol, wt, b):
    grid = N_PAD // BS2

    def body(p_ref, rd_ref, w_ref, b_ref, o_ref):
        s = (p_ref[0] + p_ref[1]) * rd_ref[...]
        o = jnp.dot(s, w_ref[...], preferred_element_type=jnp.float32)
        o_ref[...] = jnp.maximum(o + b_ref[...], 0.0)

    return pl.pallas_call(
        body,
        grid=(grid,),
        in_specs=[
            pl.BlockSpec((NC, BS2, D), lambda i: (0, i, 0)),
            pl.BlockSpec((BS2, 1), lambda i: (i, 0)),
            pl.BlockSpec((D, D), lambda i: (0, 0)),
            pl.BlockSpec((1, D), lambda i: (0, 0)),
        ],
        out_specs=pl.BlockSpec((BS2, D), lambda i: (i, 0)),
        out_shape=jax.ShapeDtypeStruct((N_PAD, D), jnp.float32),
    )(p, rdcol, wt, b)


@jax.jit
def kernel(x, edge_index, W1, b1, W2, b2):
    n = x.shape[0]
    e = edge_index.shape[1]

    eif = edge_index.reshape(2 * e).astype(jnp.int32)
    packed = lax.shift_left(eif[:e], PACK) | eif[e:]

    e_pad = -(-e // (NWK * NBUF * CHUNK)) * (NWK * NBUF * CHUNK)
    npad = e_pad - e
    pad_rows = n + (jnp.arange(npad, dtype=jnp.int32) % (N_PAD - n))
    pad_cols = jnp.arange(npad, dtype=jnp.int32) % n
    packed_p = jnp.concatenate(
        [packed, lax.shift_left(pad_rows, PACK) | pad_cols])

    z_d = jnp.zeros((N_PAD, D), dtype=jnp.float32)
    z_1 = jnp.zeros((N_PAD,), dtype=jnp.float32)

    sc1 = _make_sc_scatter(e_pad, with_deg=True)
    sc2 = _make_sc_scatter(e_pad, with_deg=False)

    p1, cnt = sc1(x, packed_p, z_d, z_1)
    rd2 = _rdeg_tc(cnt.reshape(NC, N_PAD // D, D))
    rdcol = rd2.reshape(N_PAD, 1)
    h = _mlp_tc(p1, rdcol, W1.T, b1.reshape(1, D))

    (p2,) = sc2(h, packed_p, z_d, z_1)
    out = _mlp_tc(p2, rdcol, W2.T, b2.reshape(1, D))

    return out[:n]


# speedup vs baseline: 1.2772x; 1.0851x over previous
import functools

import jax
import jax.numpy as jnp
from jax import lax
from jax.experimental import pallas as pl
from jax.experimental.pallas import tpu as pltpu
from jax.experimental.pallas import tpu_sc as plsc

N_NODES = 10000
D = 128

NC = 2
NS = 16
NWK = NC * NS

CHUNK = 32
NBUF = 8
PACK = 14
N_PAD = 10112
BS2 = 632


def _make_sc_scatter(e_pad: int, with_deg: bool):
    npw = e_pad // NWK
    nch = npw // CHUNK
    assert npw % CHUNK == 0 and nch % NBUF == 0
    rpt = N_PAD // NS

    mesh = plsc.VectorSubcoreMesh(
        core_axis_name="c", subcore_axis_name="s", num_cores=NC, num_subcores=NS)

    out_type = [jax.ShapeDtypeStruct((NC, N_PAD, D), jnp.float32)]
    scratch = [
        pltpu.VMEM((npw,), jnp.int32),
        pltpu.VMEM((NBUF, CHUNK), jnp.int32),
        pltpu.VMEM((NBUF, CHUNK), jnp.int32),
        pltpu.VMEM((NBUF, CHUNK, D), jnp.float32),
        pltpu.VMEM_SHARED((N_PAD, D), jnp.float32),
        [pltpu.SemaphoreType.DMA] * NBUF,
        [pltpu.SemaphoreType.DMA] * NBUF,
    ]
    if with_deg:
        out_type.append(jax.ShapeDtypeStruct((NC, N_PAD), jnp.float32))
        scratch += [
            pltpu.VMEM_SHARED((N_PAD,), jnp.float32),
            pltpu.VMEM((CHUNK,), jnp.float32),
        ]

    @functools.partial(
        pl.kernel,
        out_type=out_type,
        mesh=mesh,
        scratch_types=scratch,
        compiler_params=pltpu.CompilerParams(use_tc_tiling_on_sc=False),
    )
    def sc_kernel(tab, pkh, zer, zer1, *refs):
        if with_deg:
            (out, outd, pidx, colb, rowb, rowsv, acc,
             gsem, ssem, accd, ones) = refs
        else:
            out, pidx, colb, rowb, rowsv, acc, gsem, ssem = refs
        core = lax.axis_index("c")
        sub = lax.axis_index("s")
        wid = core * NS + sub
        r0 = sub * rpt

        pltpu.sync_copy(zer.at[pl.ds(r0, rpt)], acc.at[pl.ds(r0, rpt)])
        pltpu.sync_copy(pkh.at[pl.ds(wid * npw, npw)], pidx)
        if with_deg:
            @pl.when(sub == 0)
            def _zero_deg():
                pltpu.sync_copy(zer1, accd)
            for k in range(CHUNK // 16):
                ones[pl.ds(16 * k, 16)] = jnp.ones((16,), jnp.float32)
        plsc.subcore_barrier()

        def unpack(j, b):
            base = j * CHUNK
            for k in range(CHUNK // 16):
                pk = pidx[pl.ds(base + 16 * k, 16)]
                rowb[b, pl.ds(16 * k, 16)] = lax.shift_right_logical(pk, PACK)
                colb[b, pl.ds(16 * k, 16)] = lax.bitwise_and(
                    pk, (1 << PACK) - 1)

        def start_gather(b):
            pltpu.async_copy(tab.at[colb.at[b]], rowsv.at[b], gsem[b])

        def wait_gather(b):
            pltpu.make_async_copy(tab.at[colb.at[b]], rowsv.at[b],
                                  gsem[b]).wait()

        def start_scatter(b):
            pltpu.async_copy(rowsv.at[b], acc.at[rowb.at[b]], ssem[b],
                             add=True)
            if with_deg:
                pltpu.async_copy(ones, accd.at[rowb.at[b]], ssem[b], add=True)

        def wait_scatter(b):
            pltpu.make_async_copy(rowsv.at[b], acc.at[rowb.at[b]],
                                  ssem[b]).wait()
            if with_deg:
                pltpu.make_async_copy(ones, accd.at[rowb.at[b]],
                                      ssem[b]).wait()

        for j in range(NBUF - 1):
            unpack(j, j)
            start_gather(j)

        def body(i, _):
            g = NBUF * i
            for t in range(NBUF):
                j = g + t
                wait_gather(t)
                start_scatter(t)
                bp = (t - 1) % NBUF

                @pl.when(j + NBUF - 1 < nch)
                def _prefetch(j=j, t=t, bp=bp):
                    @pl.when(j > 0)
                    def _drain():
                        wait_scatter(bp)

                    unpack(j + NBUF - 1, bp)
                    start_gather(bp)
            return ()

        lax.fori_loop(0, nch // NBUF, body, ())
        for t in range(NBUF):
            wait_scatter(t)

        plsc.subcore_barrier()
        pltpu.sync_copy(acc.at[pl.ds(r0, rpt)], out.at[core].at[pl.ds(r0, rpt)])
        if with_deg:
            @pl.when(sub == 0)
            def _pub_deg():
                pltpu.sync_copy(accd, outd.at[core])

    return sc_kernel


def _rdeg_tc(cnt2d):
    nr = N_PAD // D

    def body(c_ref, rd_ref):
        rd_ref[...] = 1.0 / jnp.maximum(c_ref[0] + c_ref[1], 1.0)

    return pl.pallas_call(
        body,
        out_shape=jax.ShapeDtypeStruct((nr, D), jnp.float32),
    )(cnt2d)


def _mlp_tc(p, rdcol, wt, b):
    grid = N_PAD // BS2

    def body(p_ref, rd_ref, w_ref, b_ref, o_ref):
        s = (p_ref[0] + p_ref[1]) * rd_ref[...]
        o = jnp.dot(s, w_ref[...], preferred_element_type=jnp.float32)
        o_ref[...] = jnp.maximum(o + b_ref[...], 0.0)

    return pl.pallas_call(
        body,
        grid=(grid,),
        in_specs=[
            pl.BlockSpec((NC, BS2, D), lambda i: (0, i, 0)),
            pl.BlockSpec((BS2, 1), lambda i: (i, 0)),
            pl.BlockSpec((D, D), lambda i: (0, 0)),
            pl.BlockSpec((1, D), lambda i: (0, 0)),
        ],
        out_specs=pl.BlockSpec((BS2, D), lambda i: (i, 0)),
        out_shape=jax.ShapeDtypeStruct((N_PAD, D), jnp.float32),
    )(p, rdcol, wt, b)


@jax.jit
def kernel(x, edge_index, W1, b1, W2, b2):
    n = x.shape[0]
    e = edge_index.shape[1]

    eif = edge_index.reshape(2 * e).astype(jnp.int32)
    packed = lax.shift_left(eif[:e], PACK) | eif[e:]

    e_pad = -(-e // (NWK * NBUF * CHUNK)) * (NWK * NBUF * CHUNK)
    npad = e_pad - e
    pad_rows = n + (jnp.arange(npad, dtype=jnp.int32) % (N_PAD - n))
    pad_cols = jnp.arange(npad, dtype=jnp.int32) % n
    packed_p = jnp.concatenate(
        [packed, lax.shift_left(pad_rows, PACK) | pad_cols])

    z_d = jnp.zeros((N_PAD, D), dtype=jnp.float32)
    z_1 = jnp.zeros((N_PAD,), dtype=jnp.float32)

    sc1 = _make_sc_scatter(e_pad, with_deg=True)
    sc2 = _make_sc_scatter(e_pad, with_deg=False)

    p1, cnt = sc1(x, packed_p, z_d, z_1)
    rd2 = _rdeg_tc(cnt.reshape(NC, N_PAD // D, D))
    rdcol = rd2.reshape(N_PAD, 1)
    h = _mlp_tc(p1, rdcol, W1.T, b1.reshape(1, D))

    (p2,) = sc2(h, packed_p, z_d, z_1)
    out = _mlp_tc(p2, rdcol, W2.T, b2.reshape(1, D))

    return out[:n]
